# Initial kernel scaffold; baseline (speedup 1.0000x reference)
import jax
import jax.numpy as jnp
from jax.experimental import pallas as pl


def kernel(points, core_first, cores_mid, core_last, nodes, W1, b1, W2, b2, W3, b3):
    raise NotImplementedError("write your pallas kernel here")


# speedup vs baseline: 78.6529x; 78.6529x over previous
import functools

import jax
import jax.numpy as jnp
from jax.experimental import pallas as pl

B = 16384
D = 26
M = 64
R = 16
H = 52
BBLK = 2048
NMID = D - 2


def _tt_kernel(pT_ref, nodesT_ref, cfT_ref, cmT_ref, clT_ref,
               w1_ref, b1_ref, w2_ref, b2_ref, w3_ref, b3_ref, out_ref):
    iota_col = jax.lax.broadcasted_iota(jnp.int32, (M, 1), 0)

    def onehot_for_dim(d):
        x = pT_ref[d:d + 1, :]
        nd = nodesT_ref[:, d:d + 1]
        dist = jnp.abs(x - nd)
        minv = jnp.min(dist, axis=0, keepdims=True)
        prio = jnp.where(dist == minv, iota_col, jnp.int32(M))
        amin = jnp.min(prio, axis=0, keepdims=True)
        return (iota_col == amin).astype(jnp.float32)

    f32 = jnp.float32
    dn = (((1,), (0,)), ((), ()))

    v = jax.lax.dot_general(cfT_ref[...], onehot_for_dim(0), dn,
                            preferred_element_type=f32)
    for i in range(NMID):
        oh = onehot_for_dim(i + 1)
        sT = jax.lax.dot_general(cmT_ref[i], oh, dn,
                                 preferred_element_type=f32)
        acc = v[0:1, :] * sT[0:R, :]
        for r in range(1, R):
            acc = acc + v[r:r + 1, :] * sT[r * R:(r + 1) * R, :]
        v = acc
    lastT = jax.lax.dot_general(clT_ref[...], onehot_for_dim(D - 1), dn,
                                preferred_element_type=f32)
    tt = jnp.sum(v * lastT, axis=0)

    pT = pT_ref[...]
    h1 = jax.lax.dot_general(w1_ref[...], pT, dn, preferred_element_type=f32)
    h1 = jnp.maximum(h1 + b1_ref[...], 0.0)
    h2 = jax.lax.dot_general(w2_ref[...], h1, dn, preferred_element_type=f32)
    h2 = jnp.maximum(h2 + b2_ref[...], 0.0)
    nn = jax.lax.dot_general(w3_ref[...], h2, dn, preferred_element_type=f32)
    out_ref[...] = tt + nn[0, :] + b3_ref[0]


@jax.jit
def kernel(points, core_first, cores_mid, core_last, nodes, W1, b1, W2, b2, W3, b3):
    pT = points.T
    nodesT = nodes.T
    cfT = core_first[0].T
    cmT = jnp.transpose(cores_mid, (0, 1, 3, 2)).reshape(NMID, R * R, M)
    clT = core_last[:, :, 0]
    b1c = b1[:, None]
    b2c = b2[:, None]

    grid = (B // BBLK,)
    whole = lambda shape: pl.BlockSpec(shape, lambda i: tuple(0 for _ in shape))
    out = pl.pallas_call(
        _tt_kernel,
        grid=grid,
        in_specs=[
            pl.BlockSpec((D, BBLK), lambda i: (0, i)),
            whole((M, D)),
            whole((R, M)),
            whole((NMID, R * R, M)),
            whole((R, M)),
            whole((H, D)),
            whole((H, 1)),
            whole((H, H)),
            whole((H, 1)),
            whole((1, H)),
            whole((1,)),
        ],
        out_specs=pl.BlockSpec((BBLK,), lambda i: (i,)),
        out_shape=jax.ShapeDtypeStruct((B,), jnp.float32),
    )(pT, nodesT, cfT, cmT, clT, W1, b1c, W2, b2c, W3, b3)
    return out
